# Initial kernel scaffold; baseline (speedup 1.0000x reference)
import functools

import jax
import jax.numpy as jnp
from jax import lax
from jax.experimental import pallas as pl
from jax.experimental.pallas import tpu as pltpu
from jax.experimental.pallas import tpu_sc as plsc

_LOG2 = 0.6931471805599453
_R_CUTOFF = 5.0


def _ssp(v):
    return jnp.maximum(v, 0.0) + jnp.log1p(jnp.exp(-jnp.abs(v))) - _LOG2


def _in2f_body(x_ref, w_ref, y_ref):
    y_ref[...] = jnp.dot(x_ref[...], w_ref[...],
                         preferred_element_type=jnp.float32)


def _in2f(x, w):
    n, f_in = x.shape
    f_out = w.shape[1]
    blk = 2000
    return pl.pallas_call(
        _in2f_body,
        grid=(n // blk,),
        in_specs=[
            pl.BlockSpec((blk, f_in), lambda i: (i, 0)),
            pl.BlockSpec((f_in, f_out), lambda i: (0, 0)),
        ],
        out_specs=pl.BlockSpec((blk, f_out), lambda i: (i, 0)),
        out_shape=jax.ShapeDtypeStruct((n, f_out), jnp.float32),
    )(x, w)


def _sc_gather(y, idx2d):
    info = plsc.get_sparse_core_info()
    nc, ns = info.num_cores, info.num_subcores
    nw = nc * ns
    r_tot, c_len = idx2d.shape
    cpw = r_tot // nw
    d = y.shape[1]
    b_tot = r_tot * c_len

    mesh = plsc.VectorSubcoreMesh(core_axis_name="c", subcore_axis_name="s")

    @functools.partial(
        pl.kernel, mesh=mesh,
        out_type=jax.ShapeDtypeStruct((b_tot, d), jnp.float32),
        scratch_types=[
            pltpu.VMEM((cpw, c_len), jnp.int32),
            pltpu.VMEM((c_len, d), jnp.float32),
            pltpu.VMEM((c_len, d), jnp.float32),
            pltpu.SemaphoreType.DMA,
            pltpu.SemaphoreType.DMA,
            pltpu.SemaphoreType.DMA,
            pltpu.SemaphoreType.DMA,
        ],
    )
    def gather_kernel(y_hbm, idx_hbm, out_hbm, idx_v, b0, b1,
                      sg0, sg1, ss0, ss1):
        wid = lax.axis_index("s") * nc + lax.axis_index("c")
        base = wid * cpw
        pltpu.sync_copy(idx_hbm.at[pl.ds(base, cpw)], idx_v)

        bufs = (b0, b1)
        gsems = (sg0, sg1)
        ssems = (ss0, ss1)

        def fire_g(c, k):
            pltpu.async_copy(y_hbm.at[idx_v.at[c]], bufs[k], gsems[k])

        def wait_g(c, k):
            pltpu.make_async_copy(y_hbm.at[idx_v.at[c]], bufs[k],
                                  gsems[k]).wait()

        def fire_s(c, k):
            pltpu.async_copy(bufs[k],
                             out_hbm.at[pl.ds((base + c) * c_len, c_len)],
                             ssems[k])

        def wait_s(c, k):
            pltpu.make_async_copy(bufs[k],
                                  out_hbm.at[pl.ds((base + c) * c_len, c_len)],
                                  ssems[k]).wait()

        fire_g(0, 0)

        def body(i, carry):
            c0 = 2 * i

            @pl.when(i > 0)
            def _():
                wait_s(c0 - 1, 1)

            fire_g(c0 + 1, 1)
            wait_g(c0, 0)
            fire_s(c0, 0)
            wait_s(c0, 0)
            fire_g(c0 + 2, 0)
            wait_g(c0 + 1, 1)
            fire_s(c0 + 1, 1)
            return carry

        lax.fori_loop(0, (cpw - 1) // 2, body, 0)
        wait_s(cpw - 2, 1)
        wait_g(cpw - 1, 0)
        fire_s(cpw - 1, 0)
        wait_s(cpw - 1, 0)

    return gather_kernel(y, idx2d)


def _fused_body(dre_ref, g_ref, dr_ref, pm_ref,
                wf1_ref, bf1_ref, wf2_ref, bf2_ref,
                wfo_ref, bfo_ref, wo_ref, bo_ref, out_ref):
    a, nb = dr_ref.shape
    f = g_ref.shape[1]
    h = jnp.dot(dre_ref[...], wf1_ref[...],
                preferred_element_type=jnp.float32) + bf1_ref[...]
    h = _ssp(h)
    w = jnp.dot(h, wf2_ref[...],
                preferred_element_type=jnp.float32) + bf2_ref[...]
    m = pm_ref[...] * (dr_ref[...] < _R_CUTOFF).astype(jnp.float32)
    prod = (g_ref[...] * w).reshape(a, nb, f) * m[:, :, None]
    agg = jnp.sum(prod, axis=1)
    t = _ssp(jnp.dot(agg, wfo_ref[...],
                     preferred_element_type=jnp.float32) + bfo_ref[...])
    out_ref[...] = jnp.dot(t, wo_ref[...],
                           preferred_element_type=jnp.float32) + bo_ref[...]


def _fused(dre2, g, dr, pm, wf1, bf1, wf2, bf2, wfo, bfo, wo, bo):
    n_a, nb = dr.shape
    n_sb = dre2.shape[1]
    f = g.shape[1]
    blk_a = 250
    grid = (n_a // blk_a,)
    full = lambda shape: pl.BlockSpec(shape, lambda i, _s=shape: tuple(0 for _ in _s))
    return pl.pallas_call(
        _fused_body,
        grid=grid,
        in_specs=[
            pl.BlockSpec((blk_a * nb, n_sb), lambda i: (i, 0)),
            pl.BlockSpec((blk_a * nb, f), lambda i: (i, 0)),
            pl.BlockSpec((blk_a, nb), lambda i: (i, 0)),
            pl.BlockSpec((blk_a, nb), lambda i: (i, 0)),
            full(wf1.shape), full(bf1.shape), full(wf2.shape), full(bf2.shape),
            full(wfo.shape), full(bfo.shape), full(wo.shape), full(bo.shape),
        ],
        out_specs=pl.BlockSpec((blk_a, f), lambda i: (i, 0)),
        out_shape=jax.ShapeDtypeStruct((n_a, f), jnp.float32),
    )(dre2, g, dr, pm, wf1, bf1, wf2, bf2, wfo, bfo, wo, bo)


def kernel(x, dR, neighbors, pairwise_mask, dR_expanded,
           W_in2f, Wf1, bf1, Wf2, bf2, W_f2out, b_f2out, W_out, b_out):
    n_a, f = x.shape
    nb = neighbors.shape[1]
    n_sb = dR_expanded.shape[2]

    y = _in2f(x, W_in2f)
    idx2d = neighbors.astype(jnp.int32).reshape(n_a * nb // 80, 80)
    g = _sc_gather(y, idx2d)
    dre2 = dR_expanded.reshape(n_a * nb, n_sb)
    return _fused(
        dre2, g, dR, pairwise_mask,
        Wf1, bf1.reshape(1, -1), Wf2, bf2.reshape(1, -1),
        W_f2out, b_f2out.reshape(1, -1), W_out, b_out.reshape(1, -1))


# speedup vs baseline: 2.2278x; 2.2278x over previous
import functools

import jax
import jax.numpy as jnp
from jax import lax
from jax.experimental import pallas as pl
from jax.experimental.pallas import tpu as pltpu
from jax.experimental.pallas import tpu_sc as plsc

_LOG2 = 0.6931471805599453
_R_CUTOFF = 5.0


def _ssp(v):
    return jnp.maximum(v, 0.0) + jnp.log1p(jnp.exp(-jnp.abs(v))) - _LOG2


def _in2f_body(x_ref, w_ref, y_ref):
    y_ref[...] = jnp.dot(x_ref[...], w_ref[...],
                         preferred_element_type=jnp.float32)


def _in2f(x, w):
    n, f_in = x.shape
    f_out = w.shape[1]
    blk = 2000 if n % 2000 == 0 else n
    return pl.pallas_call(
        _in2f_body,
        grid=(n // blk,),
        in_specs=[
            pl.BlockSpec((blk, f_in), lambda i: (i, 0)),
            pl.BlockSpec((f_in, f_out), lambda i: (0, 0)),
        ],
        out_specs=pl.BlockSpec((blk, f_out), lambda i: (i, 0)),
        out_shape=jax.ShapeDtypeStruct((n, f_out), jnp.float32),
    )(x, w)


def _sc_gather(y, idx3d):
    info = plsc.get_sparse_core_info()
    nc, ns = info.num_cores, info.num_subcores
    nw = nc * ns
    nw_a, cpw, c_len = idx3d.shape
    assert nw_a == nw
    d = y.shape[1]
    b_tot = nw * cpw * c_len

    mesh = plsc.VectorSubcoreMesh(core_axis_name="c", subcore_axis_name="s")

    @functools.partial(
        pl.kernel, mesh=mesh,
        out_type=jax.ShapeDtypeStruct((b_tot, d), jnp.float32),
        scratch_types=[
            pltpu.VMEM((cpw, c_len), jnp.int32),
            pltpu.VMEM((c_len, d), jnp.float32),
            pltpu.VMEM((c_len, d), jnp.float32),
            pltpu.SemaphoreType.DMA,
            pltpu.SemaphoreType.DMA,
            pltpu.SemaphoreType.DMA,
            pltpu.SemaphoreType.DMA,
        ],
    )
    def gather_kernel(y_hbm, idx_hbm, out_hbm, idx_v, b0, b1,
                      sg0, sg1, ss0, ss1):
        wid = lax.axis_index("s") * nc + lax.axis_index("c")
        base = wid * cpw
        pltpu.sync_copy(idx_hbm.at[wid], idx_v)

        bufs = (b0, b1)
        gsems = (sg0, sg1)
        ssems = (ss0, ss1)

        def fire_g(c, k):
            pltpu.async_copy(y_hbm.at[idx_v.at[c]], bufs[k], gsems[k])

        def wait_g(c, k):
            pltpu.make_async_copy(y_hbm.at[idx_v.at[c]], bufs[k],
                                  gsems[k]).wait()

        def fire_s(c, k):
            pltpu.async_copy(bufs[k],
                             out_hbm.at[pl.ds((base + c) * c_len, c_len)],
                             ssems[k])

        def wait_s(c, k):
            pltpu.make_async_copy(bufs[k],
                                  out_hbm.at[pl.ds((base + c) * c_len, c_len)],
                                  ssems[k]).wait()

        fire_g(0, 0)

        def body(i, carry):
            c0 = 2 * i

            @pl.when(i > 0)
            def _():
                wait_s(c0 - 1, 1)

            fire_g(c0 + 1, 1)
            wait_g(c0, 0)
            fire_s(c0, 0)
            wait_s(c0, 0)
            fire_g(c0 + 2, 0)
            wait_g(c0 + 1, 1)
            fire_s(c0 + 1, 1)
            return carry

        lax.fori_loop(0, (cpw - 1) // 2, body, 0)
        wait_s(cpw - 2, 1)
        wait_g(cpw - 1, 0)
        fire_s(cpw - 1, 0)
        wait_s(cpw - 1, 0)

    return gather_kernel(y, idx3d)


def _fused_body(dre_ref, g_ref, dr_ref, pm_ref,
                wf1_ref, bf1_ref, wf2_ref, bf2_ref,
                wfo_ref, bfo_ref, wo_ref, bo_ref, out_ref):
    a, nb = dr_ref.shape
    f = g_ref.shape[1]
    h = jnp.dot(dre_ref[...], wf1_ref[...],
                preferred_element_type=jnp.float32) + bf1_ref[...]
    h = _ssp(h)
    w = jnp.dot(h, wf2_ref[...],
                preferred_element_type=jnp.float32) + bf2_ref[...]
    m = pm_ref[...] * (dr_ref[...] < _R_CUTOFF).astype(jnp.float32)
    prod = (g_ref[...] * w).reshape(a, nb, f) * m[:, :, None]
    agg = jnp.sum(prod, axis=1)
    t = _ssp(jnp.dot(agg, wfo_ref[...],
                     preferred_element_type=jnp.float32) + bfo_ref[...])
    out_ref[...] = jnp.dot(t, wo_ref[...],
                           preferred_element_type=jnp.float32) + bo_ref[...]


def _fused(dre2, g, dr, pm, wf1, bf1, wf2, bf2, wfo, bfo, wo, bo):
    n_a, nb = dr.shape
    n_sb = dre2.shape[1]
    f = g.shape[1]
    blk_a = 200
    grid = (n_a // blk_a,)
    full = lambda shape: pl.BlockSpec(shape, lambda i, _s=shape: tuple(0 for _ in _s))
    return pl.pallas_call(
        _fused_body,
        grid=grid,
        in_specs=[
            pl.BlockSpec((blk_a * nb, n_sb), lambda i: (i, 0)),
            pl.BlockSpec((blk_a * nb, f), lambda i: (i, 0)),
            pl.BlockSpec((blk_a, nb), lambda i: (i, 0)),
            pl.BlockSpec((blk_a, nb), lambda i: (i, 0)),
            full(wf1.shape), full(bf1.shape), full(wf2.shape), full(bf2.shape),
            full(wfo.shape), full(bfo.shape), full(wo.shape), full(bo.shape),
        ],
        out_specs=pl.BlockSpec((blk_a, f), lambda i: (i, 0)),
        out_shape=jax.ShapeDtypeStruct((n_a, f), jnp.float32),
    )(dre2, g, dr, pm, wf1, bf1, wf2, bf2, wfo, bfo, wo, bo)


def kernel(x, dR, neighbors, pairwise_mask, dR_expanded,
           W_in2f, Wf1, bf1, Wf2, bf2, W_f2out, b_f2out, W_out, b_out):
    n_a, f = x.shape
    nb = neighbors.shape[1]
    n_sb = dR_expanded.shape[2]

    y = _in2f(x, W_in2f)
    idx3d = neighbors.astype(jnp.int32).reshape(32, n_a * nb // (32 * 80), 80)
    g = _sc_gather(y, idx3d)
    dre2 = dR_expanded.reshape(n_a * nb, n_sb)
    return _fused(
        dre2, g, dR, pairwise_mask,
        Wf1, bf1.reshape(1, -1), Wf2, bf2.reshape(1, -1),
        W_f2out, b_f2out.reshape(1, -1), W_out, b_out.reshape(1, -1))


# speedup vs baseline: 2.6355x; 1.1830x over previous
import functools

import jax
import jax.numpy as jnp
from jax import lax
from jax.experimental import pallas as pl
from jax.experimental.pallas import tpu as pltpu
from jax.experimental.pallas import tpu_sc as plsc

_LOG2 = 0.6931471805599453
_R_CUTOFF = 5.0


def _ssp(v):
    return jnp.maximum(v, 0.0) + jnp.log1p(jnp.exp(-jnp.abs(v))) - _LOG2


def _in2f_body(x_ref, w_ref, y_ref):
    y_ref[...] = jnp.dot(x_ref[...], w_ref[...],
                         preferred_element_type=jnp.float32)


def _in2f(x, w):
    n, f_in = x.shape
    f_out = w.shape[1]
    blk = 2000 if n % 2000 == 0 else n
    return pl.pallas_call(
        _in2f_body,
        grid=(n // blk,),
        in_specs=[
            pl.BlockSpec((blk, f_in), lambda i: (i, 0)),
            pl.BlockSpec((f_in, f_out), lambda i: (0, 0)),
        ],
        out_specs=pl.BlockSpec((blk, f_out), lambda i: (i, 0)),
        out_shape=jax.ShapeDtypeStruct((n, f_out), jnp.float32),
    )(x, w)


def _sc_gather(y, idx3d):
    info = plsc.get_sparse_core_info()
    nc, ns = info.num_cores, info.num_subcores
    nw = nc * ns
    nw_a, cpw, c_len = idx3d.shape
    assert nw_a == nw
    d = y.shape[1]
    b_tot = nw * cpw * c_len

    mesh = plsc.VectorSubcoreMesh(core_axis_name="c", subcore_axis_name="s")

    @functools.partial(
        pl.kernel, mesh=mesh,
        out_type=jax.ShapeDtypeStruct((b_tot, d), jnp.float32),
        scratch_types=[
            pltpu.VMEM((cpw, c_len), jnp.int32),
            pltpu.VMEM((c_len, d), jnp.float32),
            pltpu.VMEM((c_len, d), jnp.float32),
            pltpu.SemaphoreType.DMA,
            pltpu.SemaphoreType.DMA,
            pltpu.SemaphoreType.DMA,
            pltpu.SemaphoreType.DMA,
        ],
    )
    def gather_kernel(y_hbm, idx_hbm, out_hbm, idx_v, b0, b1,
                      sg0, sg1, ss0, ss1):
        wid = lax.axis_index("s") * nc + lax.axis_index("c")
        base = wid * cpw
        pltpu.sync_copy(idx_hbm.at[wid], idx_v)

        bufs = (b0, b1)
        gsems = (sg0, sg1)
        ssems = (ss0, ss1)

        def fire_g(c, k):
            pltpu.async_copy(y_hbm.at[idx_v.at[c]], bufs[k], gsems[k])

        def wait_g(c, k):
            pltpu.make_async_copy(y_hbm.at[idx_v.at[c]], bufs[k],
                                  gsems[k]).wait()

        def fire_s(c, k):
            pltpu.async_copy(bufs[k],
                             out_hbm.at[pl.ds((base + c) * c_len, c_len)],
                             ssems[k])

        def wait_s(c, k):
            pltpu.make_async_copy(bufs[k],
                                  out_hbm.at[pl.ds((base + c) * c_len, c_len)],
                                  ssems[k]).wait()

        fire_g(0, 0)

        def body(i, carry):
            c0 = 2 * i

            @pl.when(i > 0)
            def _():
                wait_s(c0 - 1, 1)

            fire_g(c0 + 1, 1)
            wait_g(c0, 0)
            fire_s(c0, 0)
            wait_s(c0, 0)
            fire_g(c0 + 2, 0)
            wait_g(c0 + 1, 1)
            fire_s(c0 + 1, 1)
            return carry

        lax.fori_loop(0, (cpw - 1) // 2, body, 0)
        wait_s(cpw - 2, 1)
        wait_g(cpw - 1, 0)
        fire_s(cpw - 1, 0)
        wait_s(cpw - 1, 0)

    return gather_kernel(y, idx3d)


def _fused_body(dre_ref, g_ref, dr_ref, pm_ref,
                wf1_ref, bf1_ref, wf2_ref, bf2_ref,
                wfo_ref, bfo_ref, wo_ref, bo_ref, out_ref):
    a, nb = dr_ref.shape
    f = g_ref.shape[1]
    n_sb = dre_ref.shape[2]
    h = jnp.dot(dre_ref[...].reshape(a * nb, n_sb), wf1_ref[...],
                preferred_element_type=jnp.float32) + bf1_ref[...]
    h = _ssp(h)
    w = jnp.dot(h, wf2_ref[...],
                preferred_element_type=jnp.float32) + bf2_ref[...]
    m = pm_ref[...] * (dr_ref[...] < _R_CUTOFF).astype(jnp.float32)
    prod = (g_ref[...] * w).reshape(a, nb, f) * m[:, :, None]
    agg = jnp.sum(prod, axis=1)
    t = _ssp(jnp.dot(agg, wfo_ref[...],
                     preferred_element_type=jnp.float32) + bfo_ref[...])
    out_ref[...] = jnp.dot(t, wo_ref[...],
                           preferred_element_type=jnp.float32) + bo_ref[...]


def _fused(dre3, g, dr, pm, wf1, bf1, wf2, bf2, wfo, bfo, wo, bo):
    n_a, nb = dr.shape
    n_sb = dre3.shape[2]
    f = g.shape[1]
    blk_a = 200
    grid = (n_a // blk_a,)
    full = lambda shape: pl.BlockSpec(shape, lambda i, _s=shape: tuple(0 for _ in _s))
    return pl.pallas_call(
        _fused_body,
        grid=grid,
        in_specs=[
            pl.BlockSpec((blk_a, nb, n_sb), lambda i: (i, 0, 0)),
            pl.BlockSpec((blk_a * nb, f), lambda i: (i, 0)),
            pl.BlockSpec((blk_a, nb), lambda i: (i, 0)),
            pl.BlockSpec((blk_a, nb), lambda i: (i, 0)),
            full(wf1.shape), full(bf1.shape), full(wf2.shape), full(bf2.shape),
            full(wfo.shape), full(bfo.shape), full(wo.shape), full(bo.shape),
        ],
        out_specs=pl.BlockSpec((blk_a, f), lambda i: (i, 0)),
        out_shape=jax.ShapeDtypeStruct((n_a, f), jnp.float32),
    )(dre3, g, dr, pm, wf1, bf1, wf2, bf2, wfo, bfo, wo, bo)


def kernel(x, dR, neighbors, pairwise_mask, dR_expanded,
           W_in2f, Wf1, bf1, Wf2, bf2, W_f2out, b_f2out, W_out, b_out):
    n_a, f = x.shape
    nb = neighbors.shape[1]
    n_sb = dR_expanded.shape[2]

    y = _in2f(x, W_in2f)
    idx3d = neighbors.astype(jnp.int32).reshape(32, n_a * nb // (32 * 80), 80)
    g = _sc_gather(y, idx3d)
    return _fused(
        dR_expanded, g, dR, pairwise_mask,
        Wf1, bf1.reshape(1, -1), Wf2, bf2.reshape(1, -1),
        W_f2out, b_f2out.reshape(1, -1), W_out, b_out.reshape(1, -1))


# speedup vs baseline: 2.6383x; 1.0011x over previous
import functools

import jax
import jax.numpy as jnp
from jax import lax
from jax.experimental import pallas as pl
from jax.experimental.pallas import tpu as pltpu
from jax.experimental.pallas import tpu_sc as plsc

_LOG2 = 0.6931471805599453
_R_CUTOFF = 5.0


def _ssp(v):
    return jnp.maximum(v, 0.0) + jnp.log1p(jnp.exp(-jnp.abs(v))) - _LOG2


def _in2f_body(x_ref, w_ref, y_ref):
    y_ref[...] = jnp.dot(x_ref[...], w_ref[...],
                         preferred_element_type=jnp.float32)


def _in2f(x, w):
    n, f_in = x.shape
    f_out = w.shape[1]
    blk = 2000 if n % 2000 == 0 else n
    return pl.pallas_call(
        _in2f_body,
        grid=(n // blk,),
        in_specs=[
            pl.BlockSpec((blk, f_in), lambda i: (i, 0)),
            pl.BlockSpec((f_in, f_out), lambda i: (0, 0)),
        ],
        out_specs=pl.BlockSpec((blk, f_out), lambda i: (i, 0)),
        out_shape=jax.ShapeDtypeStruct((n, f_out), jnp.float32),
    )(x, w)


def _sc_gather(y, idx3d):
    info = plsc.get_sparse_core_info()
    nc, ns = info.num_cores, info.num_subcores
    nw = nc * ns
    nw_a, cpw, c_len = idx3d.shape
    assert nw_a == nw
    d = y.shape[1]
    b_tot = nw * cpw * c_len

    mesh = plsc.VectorSubcoreMesh(core_axis_name="c", subcore_axis_name="s")

    ring = 6
    ahead = 3

    @functools.partial(
        pl.kernel, mesh=mesh,
        out_type=jax.ShapeDtypeStruct((b_tot, d), jnp.float32),
        scratch_types=[
            pltpu.VMEM((cpw, c_len), jnp.int32),
        ] + [pltpu.VMEM((c_len, d), jnp.float32) for _ in range(ring)]
          + [pltpu.SemaphoreType.DMA for _ in range(2 * ring)],
    )
    def gather_kernel(y_hbm, idx_hbm, out_hbm, idx_v, *rest):
        bufs = rest[:ring]
        gsems = rest[ring:2 * ring]
        ssems = rest[2 * ring:]
        wid = lax.axis_index("s") * nc + lax.axis_index("c")
        base = wid * cpw
        pltpu.sync_copy(idx_hbm.at[wid], idx_v)

        def fire_g(c, k):
            pltpu.async_copy(y_hbm.at[idx_v.at[c]], bufs[k], gsems[k])

        def wait_g(c, k):
            pltpu.make_async_copy(y_hbm.at[idx_v.at[c]], bufs[k],
                                  gsems[k]).wait()

        def fire_s(c, k):
            pltpu.async_copy(bufs[k],
                             out_hbm.at[pl.ds((base + c) * c_len, c_len)],
                             ssems[k])

        def wait_s(c, k):
            pltpu.make_async_copy(bufs[k],
                                  out_hbm.at[pl.ds((base + c) * c_len, c_len)],
                                  ssems[k]).wait()

        for k in range(ahead):
            fire_g(k, k)

        n_iter = (cpw + ring - 1) // ring

        def body(j, carry):
            c0 = j * ring
            for k in range(ring):
                c = c0 + k

                @pl.when(c < cpw)
                def _(c=c, k=k):
                    wait_g(c, k)
                    fire_s(c, k)

                kg = (k + ahead) % ring

                @pl.when(c + ahead < cpw)
                def _(c=c, kg=kg):
                    @pl.when(c >= ring - ahead)
                    def _():
                        wait_s(c + ahead - ring, kg)
                    fire_g(c + ahead, kg)
            return carry

        lax.fori_loop(0, n_iter, body, 0)
        for c in range(cpw - ahead, cpw):
            wait_s(c, c % ring)

    return gather_kernel(y, idx3d)


def _fused_body(dre_ref, g_ref, dr_ref, pm_ref,
                wf1_ref, bf1_ref, wf2_ref, bf2_ref,
                wfo_ref, bfo_ref, wo_ref, bo_ref, out_ref):
    a, nb = dr_ref.shape
    f = g_ref.shape[1]
    n_sb = dre_ref.shape[2]
    h = jnp.dot(dre_ref[...].reshape(a * nb, n_sb), wf1_ref[...],
                preferred_element_type=jnp.float32) + bf1_ref[...]
    h = _ssp(h)
    w = jnp.dot(h, wf2_ref[...],
                preferred_element_type=jnp.float32) + bf2_ref[...]
    m = pm_ref[...] * (dr_ref[...] < _R_CUTOFF).astype(jnp.float32)
    prod = (g_ref[...] * w).reshape(a, nb, f) * m[:, :, None]
    agg = jnp.sum(prod, axis=1)
    t = _ssp(jnp.dot(agg, wfo_ref[...],
                     preferred_element_type=jnp.float32) + bfo_ref[...])
    out_ref[...] = jnp.dot(t, wo_ref[...],
                           preferred_element_type=jnp.float32) + bo_ref[...]


def _fused(dre3, g, dr, pm, wf1, bf1, wf2, bf2, wfo, bfo, wo, bo):
    n_a, nb = dr.shape
    n_sb = dre3.shape[2]
    f = g.shape[1]
    blk_a = 200
    grid = (n_a // blk_a,)
    full = lambda shape: pl.BlockSpec(shape, lambda i, _s=shape: tuple(0 for _ in _s))
    return pl.pallas_call(
        _fused_body,
        grid=grid,
        in_specs=[
            pl.BlockSpec((blk_a, nb, n_sb), lambda i: (i, 0, 0)),
            pl.BlockSpec((blk_a * nb, f), lambda i: (i, 0)),
            pl.BlockSpec((blk_a, nb), lambda i: (i, 0)),
            pl.BlockSpec((blk_a, nb), lambda i: (i, 0)),
            full(wf1.shape), full(bf1.shape), full(wf2.shape), full(bf2.shape),
            full(wfo.shape), full(bfo.shape), full(wo.shape), full(bo.shape),
        ],
        out_specs=pl.BlockSpec((blk_a, f), lambda i: (i, 0)),
        out_shape=jax.ShapeDtypeStruct((n_a, f), jnp.float32),
    )(dre3, g, dr, pm, wf1, bf1, wf2, bf2, wfo, bfo, wo, bo)


def kernel(x, dR, neighbors, pairwise_mask, dR_expanded,
           W_in2f, Wf1, bf1, Wf2, bf2, W_f2out, b_f2out, W_out, b_out):
    n_a, f = x.shape
    nb = neighbors.shape[1]
    n_sb = dR_expanded.shape[2]

    y = _in2f(x, W_in2f)
    idx3d = neighbors.astype(jnp.int32).reshape(32, n_a * nb // (32 * 80), 80)
    g = _sc_gather(y, idx3d)
    return _fused(
        dR_expanded, g, dR, pairwise_mask,
        Wf1, bf1.reshape(1, -1), Wf2, bf2.reshape(1, -1),
        W_f2out, b_f2out.reshape(1, -1), W_out, b_out.reshape(1, -1))


# speedup vs baseline: 2.7599x; 1.0461x over previous
import functools

import jax
import jax.numpy as jnp
from jax import lax
from jax.experimental import pallas as pl
from jax.experimental.pallas import tpu as pltpu
from jax.experimental.pallas import tpu_sc as plsc

_LOG2 = 0.6931471805599453
_R_CUTOFF = 5.0


def _sp(v):
    return jnp.maximum(v, 0.0) + jnp.log(1.0 + jnp.exp(-jnp.abs(v)))


def _in2f_body(x_ref, w_ref, y_ref):
    y_ref[...] = jnp.dot(x_ref[...], w_ref[...],
                         preferred_element_type=jnp.float32)


def _in2f(x, w):
    n, f_in = x.shape
    f_out = w.shape[1]
    blk = 2000 if n % 2000 == 0 else n
    return pl.pallas_call(
        _in2f_body,
        grid=(n // blk,),
        in_specs=[
            pl.BlockSpec((blk, f_in), lambda i: (i, 0)),
            pl.BlockSpec((f_in, f_out), lambda i: (0, 0)),
        ],
        out_specs=pl.BlockSpec((blk, f_out), lambda i: (i, 0)),
        out_shape=jax.ShapeDtypeStruct((n, f_out), jnp.float32),
    )(x, w)


def _sc_gather(y, idx3d):
    info = plsc.get_sparse_core_info()
    nc, ns = info.num_cores, info.num_subcores
    nw = nc * ns
    nw_a, cpw, c_len = idx3d.shape
    assert nw_a == nw
    d = y.shape[1]
    b_tot = nw * cpw * c_len

    mesh = plsc.VectorSubcoreMesh(core_axis_name="c", subcore_axis_name="s")

    ring = 6
    ahead = 3

    @functools.partial(
        pl.kernel, mesh=mesh,
        out_type=jax.ShapeDtypeStruct((b_tot, d), y.dtype),
        scratch_types=[
            pltpu.VMEM((cpw, c_len), jnp.int32),
        ] + [pltpu.VMEM((c_len, d), y.dtype) for _ in range(ring)]
          + [pltpu.SemaphoreType.DMA for _ in range(2 * ring)],
    )
    def gather_kernel(y_hbm, idx_hbm, out_hbm, idx_v, *rest):
        bufs = rest[:ring]
        gsems = rest[ring:2 * ring]
        ssems = rest[2 * ring:]
        wid = lax.axis_index("s") * nc + lax.axis_index("c")
        base = wid * cpw
        pltpu.sync_copy(idx_hbm.at[wid], idx_v)

        def fire_g(c, k):
            pltpu.async_copy(y_hbm.at[idx_v.at[c]], bufs[k], gsems[k])

        def wait_g(c, k):
            pltpu.make_async_copy(y_hbm.at[idx_v.at[c]], bufs[k],
                                  gsems[k]).wait()

        def fire_s(c, k):
            pltpu.async_copy(bufs[k],
                             out_hbm.at[pl.ds((base + c) * c_len, c_len)],
                             ssems[k])

        def wait_s(c, k):
            pltpu.make_async_copy(bufs[k],
                                  out_hbm.at[pl.ds((base + c) * c_len, c_len)],
                                  ssems[k]).wait()

        for k in range(ahead):
            fire_g(k, k)

        n_iter = (cpw + ring - 1) // ring

        def body(j, carry):
            c0 = j * ring
            for k in range(ring):
                c = c0 + k

                @pl.when(c < cpw)
                def _(c=c, k=k):
                    wait_g(c, k)
                    fire_s(c, k)

                kg = (k + ahead) % ring

                @pl.when(c + ahead < cpw)
                def _(c=c, kg=kg):
                    @pl.when(c >= ring - ahead)
                    def _():
                        wait_s(c + ahead - ring, kg)
                    fire_g(c + ahead, kg)
            return carry

        lax.fori_loop(0, n_iter, body, 0)
        for c in range(cpw - ahead, cpw):
            wait_s(c, c % ring)

    return gather_kernel(y, idx3d)


def _fused_body(dre_ref, g_ref, dr_ref, pm_ref,
                wf1_ref, bf1_ref, wf2_ref, bf2_ref,
                wfo_ref, bfo_ref, wo_ref, bo_ref, out_ref):
    a, nb = dr_ref.shape
    f = g_ref.shape[1]
    n_sb = dre_ref.shape[2]
    h = jnp.dot(dre_ref[...].reshape(a * nb, n_sb), wf1_ref[...],
                preferred_element_type=jnp.float32) + bf1_ref[...]
    h = _sp(h)
    w = jnp.dot(h, wf2_ref[...],
                preferred_element_type=jnp.float32) + bf2_ref[...]
    m = pm_ref[...] * (dr_ref[...] < _R_CUTOFF).astype(jnp.float32)
    prod = (g_ref[...] * w).reshape(a, nb, f) * m[:, :, None]
    agg = jnp.sum(prod, axis=1)
    t = _sp(jnp.dot(agg, wfo_ref[...],
                     preferred_element_type=jnp.float32) + bfo_ref[...])
    out_ref[...] = jnp.dot(t, wo_ref[...],
                           preferred_element_type=jnp.float32) + bo_ref[...]


def _fused(dre3, g, dr, pm, wf1, bf1, wf2, bf2, wfo, bfo, wo, bo):
    n_a, nb = dr.shape
    n_sb = dre3.shape[2]
    f = g.shape[1]
    blk_a = 200
    grid = (n_a // blk_a,)
    full = lambda shape: pl.BlockSpec(shape, lambda i, _s=shape: tuple(0 for _ in _s))
    return pl.pallas_call(
        _fused_body,
        grid=grid,
        in_specs=[
            pl.BlockSpec((blk_a, nb, n_sb), lambda i: (i, 0, 0)),
            pl.BlockSpec((blk_a * nb, f), lambda i: (i, 0)),
            pl.BlockSpec((blk_a, nb), lambda i: (i, 0)),
            pl.BlockSpec((blk_a, nb), lambda i: (i, 0)),
            full(wf1.shape), full(bf1.shape), full(wf2.shape), full(bf2.shape),
            full(wfo.shape), full(bfo.shape), full(wo.shape), full(bo.shape),
        ],
        out_specs=pl.BlockSpec((blk_a, f), lambda i: (i, 0)),
        out_shape=jax.ShapeDtypeStruct((n_a, f), jnp.float32),
    )(dre3, g, dr, pm, wf1, bf1, wf2, bf2, wfo, bfo, wo, bo)


def kernel(x, dR, neighbors, pairwise_mask, dR_expanded,
           W_in2f, Wf1, bf1, Wf2, bf2, W_f2out, b_f2out, W_out, b_out):
    n_a, f = x.shape
    nb = neighbors.shape[1]
    n_sb = dR_expanded.shape[2]

    y = _in2f(x, W_in2f)
    idx3d = neighbors.astype(jnp.int32).reshape(32, n_a * nb // (32 * 80), 80)
    g = _sc_gather(y, idx3d)
    bf2_adj = bf2 - _LOG2 * jnp.sum(Wf2, axis=0)
    b_out_adj = b_out - _LOG2 * jnp.sum(W_out, axis=0)
    return _fused(
        dR_expanded, g, dR, pairwise_mask,
        Wf1, bf1.reshape(1, -1), Wf2, bf2_adj.reshape(1, -1),
        W_f2out, b_f2out.reshape(1, -1), W_out, b_out_adj.reshape(1, -1))
